# baseline (speedup 1.0000x reference)
import functools

import jax
import jax.numpy as jnp
from jax import lax
from jax.experimental import pallas as pl
from jax.experimental.pallas import tpu as pltpu
from jax.experimental.pallas import tpu_sc as plsc

_NC = 2
_NS = 16
_NW = _NC * _NS
_CHUNK = 16


def _gather_body(tt_hbm, idx_hbm, xt_hbm, idx_v, slab_v, xt_v, sem):
    d = tt_hbm.shape[0]
    b_per_w = idx_v.shape[0]
    wid = lax.axis_index("s") * _NC + lax.axis_index("c")
    base = wid * b_per_w
    pltpu.sync_copy(idx_hbm.at[pl.ds(base, b_per_w)], idx_v)

    iota = lax.iota(jnp.int32, 16)

    def round_body(r, carry):
        j0 = r * _CHUNK
        vblk = (idx_v[pl.ds(j0, 16)] // 128) * 128
        handles = []
        for t in range(_CHUNK):
            blk = pl.multiple_of(vblk[t], 128)
            handles.append(
                pltpu.async_copy(
                    tt_hbm.at[:, pl.ds(blk, 128)], slab_v.at[t], sem
                )
            )
        for h in handles:
            h.wait()
        rem = idx_v[pl.ds(j0, 16)] & 127
        for c in range(d):
            cv = jnp.full((16,), c, jnp.int32)
            xt_v[c, pl.ds(j0, 16)] = plsc.load_gather(
                slab_v, [iota, cv, rem]
            )
        return carry

    lax.fori_loop(0, b_per_w // _CHUNK, round_body, 0)
    pltpu.sync_copy(xt_v, xt_hbm.at[:, pl.ds(base, b_per_w)])


@functools.partial(jax.jit, static_argnames=("b", "d"))
def _sc_gather_t(tt, idx, b, d):
    b_per_w = b // _NW
    mesh = plsc.VectorSubcoreMesh(
        core_axis_name="c", subcore_axis_name="s", num_cores=_NC,
        num_subcores=_NS,
    )
    return pl.kernel(
        _gather_body,
        out_type=jax.ShapeDtypeStruct((d, b), jnp.float32),
        mesh=mesh,
        scratch_types=[
            pltpu.VMEM((b_per_w,), jnp.int32),
            pltpu.VMEM((_CHUNK, d, 128), jnp.float32),
            pltpu.VMEM((d, b_per_w), jnp.float32),
            pltpu.SemaphoreType.DMA,
        ],
        compiler_params=pltpu.CompilerParams(needs_layout_passes=False),
    )(tt, idx)


def _cov_body(xa_ref, xb_ref, o_ref):
    o_ref[...] = lax.dot_general(
        xa_ref[...], xb_ref[...],
        dimension_numbers=(((0,), (0,)), ((), ())),
        preferred_element_type=jnp.float32,
    )


@functools.partial(jax.jit, static_argnames=("bm",))
def _tc_cov_t(xt, bm):
    d, b = xt.shape
    return pl.pallas_call(
        _cov_body,
        grid=(b // bm,),
        in_specs=[
            pl.BlockSpec((d, bm), lambda i: (0, i)),
            pl.BlockSpec((d, b), lambda i: (0, 0)),
        ],
        out_specs=pl.BlockSpec((bm, b), lambda i: (i, 0)),
        out_shape=jax.ShapeDtypeStruct((b, b), jnp.float32),
        compiler_params=pltpu.CompilerParams(
            dimension_semantics=("arbitrary",),
        ),
    )(xt, xt)


def kernel(states, table):
    b = states.shape[0]
    d = table.shape[1]
    idx = states.reshape(b).astype(jnp.int32)
    xt = _sc_gather_t(table.T, idx, b, d)
    cov = _tc_cov_t(xt, 512)
    return (xt.T, cov)
